# baseline (speedup 1.0000x reference)
import functools

import jax
import jax.numpy as jnp
from jax import lax
from jax.experimental import pallas as pl
from jax.experimental.pallas import tpu as pltpu
from jax.experimental.pallas import tpu_sc as plsc

B, S, V = 16, 16, 100000
N = B * S
LANES = 16
NWORK = N // LANES

_LN2 = 0.6931471805599453
_SQRT2 = 1.4142135623730951


def _neg_log(x):
    bits = lax.bitcast_convert_type(x, jnp.int32)
    e = ((bits >> 23) & 0xFF) - 127
    m = lax.bitcast_convert_type((bits & 0x007FFFFF) | 0x3F800000, jnp.float32)
    big = m > _SQRT2
    m = jnp.where(big, m * 0.5, m)
    e = jnp.where(big, e + 1, e)
    s = (m - 1.0) / (m + 1.0)
    z = s * s
    poly = 2.0 * s * (1.0 + z * (1.0 / 3.0 + z * (1.0 / 5.0 + z * (1.0 / 7.0 + z * (1.0 / 9.0)))))
    return -(poly + e.astype(jnp.float32) * _LN2)


def _lane_sum(v):
    lane = lax.iota(jnp.int32, LANES)
    dnums = lax.GatherDimensionNumbers(
        offset_dims=(), collapsed_slice_dims=(0,), start_index_map=(0,))
    for sh in (8, 4, 2, 1):
        perm = jnp.reshape(lane ^ sh, (LANES, 1))
        v = v + lax.gather(v, perm, dnums, (1,),
                           mode=lax.GatherScatterMode.PROMISE_IN_BOUNDS)
    return v


def _loss_kernel(out_hbm, tgt_hbm, res_hbm, tgt_v, val_i, val_v, stage_v,
                 acc_v, shared, sem):
    c = lax.axis_index("c")
    s = lax.axis_index("s")

    @pl.when(c == 0)
    def _gather_and_partial():
        base = s * LANES
        pltpu.sync_copy(tgt_hbm.at[s], tgt_v)
        tgt = tgt_v[...]
        k = base + lax.iota(jnp.int32, LANES)
        flat = k * V + tgt
        pltpu.async_copy(out_hbm.at[flat >> 1], val_i, sem).wait()
        w = val_i[...]
        half = jnp.where((flat & 1) == 1, (w >> 16) & 0xFFFF, w & 0xFFFF)
        x = lax.bitcast_convert_type(half << 16, jnp.float32)
        val_v[...] = _neg_log(x)
        pltpu.sync_copy(val_v, shared.at[pl.ds(s * LANES, LANES)])

    plsc.subcore_barrier()

    @pl.when((c == 0) & (s == 0))
    def _reduce():
        pltpu.sync_copy(shared, stage_v)
        acc = jnp.zeros((LANES,), jnp.float32)
        for w in range(NWORK):
            acc = acc + stage_v[pl.ds(w * LANES, LANES)]
        acc_v[...] = _lane_sum(acc)
        pltpu.sync_copy(acc_v, res_hbm)


@jax.jit
def _loss(out_flat, tgt_flat):
    mesh = plsc.VectorSubcoreMesh(
        core_axis_name="c", subcore_axis_name="s", num_cores=1)
    run = functools.partial(
        pl.kernel,
        mesh=mesh,
        out_type=jax.ShapeDtypeStruct((LANES,), jnp.float32),
        scratch_types=[
            pltpu.VMEM((LANES,), jnp.int32),
            pltpu.VMEM((LANES,), jnp.int32),
            pltpu.VMEM((LANES,), jnp.float32),
            pltpu.VMEM((N,), jnp.float32),
            pltpu.VMEM((LANES,), jnp.float32),
            pltpu.VMEM_SHARED((N,), jnp.float32),
            pltpu.SemaphoreType.DMA,
        ],
    )(_loss_kernel)
    return run(out_flat, tgt_flat)


def kernel(output, target):
    packed = jax.lax.bitcast_convert_type(
        output.astype(jnp.bfloat16).reshape(-1, 2), jnp.int32)
    tgt2d = target.astype(jnp.int32)
    res = _loss(packed, tgt2d)
    return res[0].reshape(())


# speedup vs baseline: 2.5164x; 2.5164x over previous
import functools

import jax
import jax.numpy as jnp
from jax import lax
from jax.experimental import pallas as pl
from jax.experimental.pallas import tpu as pltpu
from jax.experimental.pallas import tpu_sc as plsc

B, S, V = 16, 16, 100000
N = B * S
LANES = 16
NWORK = N // LANES

_LN2 = 0.6931471805599453
_SQRT2 = 1.4142135623730951


def _neg_log(x):
    bits = lax.bitcast_convert_type(x, jnp.int32)
    e = ((bits >> 23) & 0xFF) - 127
    m = lax.bitcast_convert_type((bits & 0x007FFFFF) | 0x3F800000, jnp.float32)
    big = m > _SQRT2
    m = jnp.where(big, m * 0.5, m)
    e = jnp.where(big, e + 1, e)
    s = (m - 1.0) / (m + 1.0)
    z = s * s
    poly = 2.0 * s * (1.0 + z * (1.0 / 3.0 + z * (1.0 / 5.0 + z * (1.0 / 7.0 + z * (1.0 / 9.0)))))
    return -(poly + e.astype(jnp.float32) * _LN2)


def _lane_sum(v):
    lane = lax.iota(jnp.int32, LANES)
    dnums = lax.GatherDimensionNumbers(
        offset_dims=(), collapsed_slice_dims=(0,), start_index_map=(0,))
    for sh in (8, 4, 2, 1):
        perm = jnp.reshape(lane ^ sh, (LANES, 1))
        v = v + lax.gather(v, perm, dnums, (1,),
                           mode=lax.GatherScatterMode.PROMISE_IN_BOUNDS)
    return v


def _loss_kernel(out_hbm, tgt_hbm, res_hbm, tgt_v, val_i, val_v, stage_v,
                 acc_v, shared, sem):
    c = lax.axis_index("c")
    s = lax.axis_index("s")

    @pl.when(c == 0)
    def _gather_and_partial():
        base = s * LANES
        pltpu.sync_copy(tgt_hbm.at[s], tgt_v)
        tgt = tgt_v[...]
        k = base + lax.iota(jnp.int32, LANES)
        flat = k * V + tgt
        pltpu.async_copy(out_hbm.at[flat >> 1], val_i, sem).wait()
        w = val_i[...]
        half = jnp.where((flat & 1) == 1, (w >> 16) & 0xFFFF, w & 0xFFFF)
        x = lax.bitcast_convert_type(half << 16, jnp.float32)
        val_v[...] = _neg_log(x)
        pltpu.sync_copy(val_v, shared.at[pl.ds(s * LANES, LANES)])

    plsc.subcore_barrier()

    @pl.when((c == 0) & (s == 0))
    def _reduce():
        pltpu.sync_copy(shared, stage_v)
        acc = jnp.zeros((LANES,), jnp.float32)
        for w in range(NWORK):
            acc = acc + stage_v[pl.ds(w * LANES, LANES)]
        acc_v[...] = _lane_sum(acc)
        pltpu.sync_copy(acc_v, res_hbm)


@jax.jit
def _loss(out_flat, tgt_flat):
    mesh = plsc.VectorSubcoreMesh(
        core_axis_name="c", subcore_axis_name="s", num_cores=1)
    run = functools.partial(
        pl.kernel,
        mesh=mesh,
        out_type=jax.ShapeDtypeStruct((LANES,), jnp.float32),
        scratch_types=[
            pltpu.VMEM((LANES,), jnp.int32),
            pltpu.VMEM((LANES,), jnp.int32),
            pltpu.VMEM((LANES,), jnp.float32),
            pltpu.VMEM((N,), jnp.float32),
            pltpu.VMEM((LANES,), jnp.float32),
            pltpu.VMEM_SHARED((N,), jnp.float32),
            pltpu.SemaphoreType.DMA,
        ],
    )(_loss_kernel)
    return run(out_flat, tgt_flat)


def kernel(output, target):
    bits = jax.lax.bitcast_convert_type(
        output.astype(jnp.bfloat16), jnp.uint16)
    even = bits[:, :, 0::2].astype(jnp.uint32)
    odd = bits[:, :, 1::2].astype(jnp.uint32)
    packed = jax.lax.bitcast_convert_type(
        (even | (odd << 16)).reshape(-1), jnp.int32)
    tgt2d = target.astype(jnp.int32)
    res = _loss(packed, tgt2d)
    return res[0].reshape(())


# speedup vs baseline: 57.4193x; 22.8185x over previous
import functools

import jax
import jax.numpy as jnp
from jax import lax
from jax.experimental import pallas as pl
from jax.experimental.pallas import tpu as pltpu
from jax.experimental.pallas import tpu_sc as plsc

B, S, V = 16, 16, 100000
N = B * S
LANES = 16
NWORK = N // LANES

_LN2 = 0.6931471805599453
_SQRT2 = 1.4142135623730951


def _neg_log(x):
    bits = lax.bitcast_convert_type(x, jnp.int32)
    e = ((bits >> 23) & 0xFF) - 127
    m = lax.bitcast_convert_type((bits & 0x007FFFFF) | 0x3F800000, jnp.float32)
    big = m > _SQRT2
    m = jnp.where(big, m * 0.5, m)
    e = jnp.where(big, e + 1, e)
    s = (m - 1.0) / (m + 1.0)
    z = s * s
    poly = 2.0 * s * (1.0 + z * (1.0 / 3.0 + z * (1.0 / 5.0 + z * (1.0 / 7.0 + z * (1.0 / 9.0)))))
    return -(poly + e.astype(jnp.float32) * _LN2)


def _lane_sum(v):
    lane = lax.iota(jnp.int32, LANES)
    dnums = lax.GatherDimensionNumbers(
        offset_dims=(), collapsed_slice_dims=(0,), start_index_map=(0,))
    for sh in (8, 4, 2, 1):
        perm = jnp.reshape(lane ^ sh, (LANES, 1))
        v = v + lax.gather(v, perm, dnums, (1,),
                           mode=lax.GatherScatterMode.PROMISE_IN_BOUNDS)
    return v


def _loss_kernel(out_hbm, tgt_hbm, res_hbm, tgt_v, val_v, stage_v, acc_v,
                 shared, sem):
    c = lax.axis_index("c")
    s = lax.axis_index("s")

    @pl.when(c == 0)
    def _gather_and_partial():
        base = s * LANES
        pltpu.sync_copy(tgt_hbm.at[s], tgt_v)
        tgt = tgt_v[...]
        k = base + lax.iota(jnp.int32, LANES)
        idx = k * V + tgt
        pltpu.async_copy(out_hbm.at[idx], val_v, sem).wait()
        val_v[...] = _neg_log(val_v[...])
        pltpu.sync_copy(val_v, shared.at[pl.ds(s * LANES, LANES)])

    plsc.subcore_barrier()

    @pl.when((c == 0) & (s == 0))
    def _reduce():
        pltpu.sync_copy(shared, stage_v)
        acc = jnp.zeros((LANES,), jnp.float32)
        for w in range(NWORK):
            acc = acc + stage_v[pl.ds(w * LANES, LANES)]
        acc_v[...] = _lane_sum(acc)
        pltpu.sync_copy(acc_v, res_hbm)


@jax.jit
def _loss(out_flat, tgt_flat):
    mesh = plsc.VectorSubcoreMesh(
        core_axis_name="c", subcore_axis_name="s", num_cores=1)
    run = functools.partial(
        pl.kernel,
        mesh=mesh,
        out_type=jax.ShapeDtypeStruct((LANES,), jnp.float32),
        scratch_types=[
            pltpu.VMEM((LANES,), jnp.int32),
            pltpu.VMEM((LANES,), jnp.float32),
            pltpu.VMEM((N,), jnp.float32),
            pltpu.VMEM((LANES,), jnp.float32),
            pltpu.VMEM_SHARED((N,), jnp.float32),
            pltpu.SemaphoreType.DMA,
        ],
    )(_loss_kernel)
    return run(out_flat, tgt_flat)


def kernel(output, target):
    out_flat = output.reshape(-1)
    tgt2d = target.astype(jnp.int32)
    res = _loss(out_flat, tgt2d)
    return res[0].reshape(())
